# baseline (speedup 1.0000x reference)
import jax
import jax.numpy as jnp
from jax.experimental import pallas as pl
from jax.experimental.pallas import tpu as pltpu

_TAU = 16
_NX, _NY, _D = 24, 24, 768
_D3 = 256
_CH = 4
_NBT = 64
_NCHUNK = _NBT // _CH
_NBUF = 3


def _pipe_kernel(tok_hbm, x_ref, y_ref, t_ref, out_hbm, *scratch):
    ins = scratch[0:_NBUF]
    outs = scratch[_NBUF:2 * _NBUF]
    sis = scratch[2 * _NBUF:3 * _NBUF]
    sos = scratch[3 * _NBUF:4 * _NBUF]
    x = x_ref[...]
    y = y_ref[...]

    def in_copy(c, b):
        return pltpu.make_async_copy(
            tok_hbm.at[pl.ds(c * _CH, _CH)], ins[b], sis[b])

    def out_copy(c, b):
        return pltpu.make_async_copy(
            outs[b], out_hbm.at[pl.ds(c * _CH, _CH)], sos[b])

    for b in range(_NBUF):
        in_copy(b, b).start()
    for c in range(_NCHUNK):
        b = c % _NBUF
        in_copy(c, b).wait()
        if c >= _NBUF:
            out_copy(c - _NBUF, b).wait()
        for u in range(_CH):
            ti = (c * _CH + u) % _TAU
            tok = ins[b][u]
            outs[b][u, :, :, 0:_D3] = tok[:, :, 0:_D3] + x[:, None, :]
            outs[b][u, :, :, _D3:2 * _D3] = tok[:, :, _D3:2 * _D3] + y[None, :, :]
            outs[b][u, :, :, 2 * _D3:3 * _D3] = tok[:, :, 2 * _D3:3 * _D3] + t_ref[ti]
        if c + _NBUF < _NCHUNK:
            in_copy(c + _NBUF, b).start()
        out_copy(c, b).start()
    for c in range(_NCHUNK - _NBUF, _NCHUNK):
        out_copy(c, c % _NBUF).wait()


def kernel(tokens, n_x, n_y, x_emb, y_emb, t_emb):
    B, tau, N, d = tokens.shape
    nx = x_emb.shape[0]
    ny = y_emb.shape[0]
    tok4 = tokens.reshape(B * tau, nx, ny, d)

    out4 = pl.pallas_call(
        _pipe_kernel,
        grid=(1,),
        in_specs=[
            pl.BlockSpec(memory_space=pltpu.MemorySpace.HBM),
            pl.BlockSpec((nx, _D3), lambda i: (0, 0)),
            pl.BlockSpec((ny, _D3), lambda i: (0, 0)),
            pl.BlockSpec((tau, 1, _D3), lambda i: (0, 0, 0)),
        ],
        out_specs=pl.BlockSpec(memory_space=pltpu.MemorySpace.HBM),
        out_shape=jax.ShapeDtypeStruct((B * tau, nx, ny, d), tokens.dtype),
        scratch_shapes=(
            [pltpu.VMEM((_CH, nx, ny, d), jnp.float32) for _ in range(2 * _NBUF)]
            + [pltpu.SemaphoreType.DMA for _ in range(2 * _NBUF)]
        ),
        compiler_params=pltpu.CompilerParams(
            vmem_limit_bytes=128 * 1024 * 1024,
        ),
    )(tok4, x_emb, y_emb, t_emb.reshape(tau, 1, _D3))

    return out4.reshape(B, tau, N, d)


# speedup vs baseline: 1.0060x; 1.0060x over previous
import jax
import jax.numpy as jnp
from jax.experimental import pallas as pl
from jax.experimental.pallas import tpu as pltpu

_D_MODEL = 768
_D3 = _D_MODEL // 3
_TAU_BLK = 8


def _embed_add_kernel(tok_ref, x_ref, y_ref, t_ref, out_ref):
    x = x_ref[...]
    y = y_ref[...]
    d = _D3
    for i in range(_TAU_BLK):
        tok = tok_ref[0, i]
        out_ref[0, i, :, :, 0:d] = tok[:, :, 0:d] + x[:, None, :]
        out_ref[0, i, :, :, d:2 * d] = tok[:, :, d:2 * d] + y[None, :, :]
        out_ref[0, i, :, :, 2 * d:3 * d] = tok[:, :, 2 * d:3 * d] + t_ref[i]


def kernel(tokens, n_x, n_y, x_emb, y_emb, t_emb):
    B, tau, N, d = tokens.shape
    nx = x_emb.shape[0]
    ny = y_emb.shape[0]
    tok5 = tokens.reshape(B, tau, nx, ny, d)

    out5 = pl.pallas_call(
        _embed_add_kernel,
        grid=(B * tau // _TAU_BLK,),
        in_specs=[
            pl.BlockSpec((1, _TAU_BLK, nx, ny, d), lambda g: (g // 2, g % 2, 0, 0, 0)),
            pl.BlockSpec((nx, _D3), lambda g: (0, 0)),
            pl.BlockSpec((ny, _D3), lambda g: (0, 0)),
            pl.BlockSpec((_TAU_BLK, 1, _D3), lambda g: (g % 2, 0, 0)),
        ],
        out_specs=pl.BlockSpec((1, _TAU_BLK, nx, ny, d), lambda g: (g // 2, g % 2, 0, 0, 0)),
        out_shape=jax.ShapeDtypeStruct((B, tau, nx, ny, d), tokens.dtype),
        compiler_params=pltpu.CompilerParams(
            dimension_semantics=("parallel",),
            vmem_limit_bytes=128 * 1024 * 1024,
        ),
    )(tok5, x_emb, y_emb, t_emb.reshape(tau, 1, _D3))

    return out5.reshape(B, tau, N, d)


# speedup vs baseline: 1.0078x; 1.0018x over previous
import jax
from jax.experimental import pallas as pl
from jax.experimental.pallas import tpu as pltpu

_D_MODEL = 768
_D3 = _D_MODEL // 3
_TAU_BLK = 8


def _embed_add_kernel(tok_ref, x_ref, y_ref, t_ref, out_ref):
    x = x_ref[...]
    y = y_ref[...]
    d = _D3
    for i in range(_TAU_BLK):
        tok = tok_ref[0, i]
        out_ref[0, i, :, :, 0:d] = tok[:, :, 0:d] + x[:, None, :]
        out_ref[0, i, :, :, d:2 * d] = tok[:, :, d:2 * d] + y[None, :, :]
        out_ref[0, i, :, :, 2 * d:3 * d] = tok[:, :, 2 * d:3 * d] + t_ref[i]


def kernel(tokens, n_x, n_y, x_emb, y_emb, t_emb):
    B, tau, N, d = tokens.shape
    nx = x_emb.shape[0]
    ny = y_emb.shape[0]
    tok5 = tokens.reshape(B, tau, nx, ny, d)
    ntb = tau // _TAU_BLK

    out5 = pl.pallas_call(
        _embed_add_kernel,
        grid=(B * ntb,),
        in_specs=[
            pl.BlockSpec((1, _TAU_BLK, nx, ny, d),
                         lambda g: (g // ntb, g % ntb, 0, 0, 0)),
            pl.BlockSpec((nx, _D3), lambda g: (0, 0)),
            pl.BlockSpec((ny, _D3), lambda g: (0, 0)),
            pl.BlockSpec((_TAU_BLK, 1, _D3), lambda g: (g % ntb, 0, 0)),
        ],
        out_specs=pl.BlockSpec((1, _TAU_BLK, nx, ny, d),
                               lambda g: (g // ntb, g % ntb, 0, 0, 0)),
        out_shape=jax.ShapeDtypeStruct((B, tau, nx, ny, d), tokens.dtype),
        compiler_params=pltpu.CompilerParams(
            dimension_semantics=("parallel",),
            vmem_limit_bytes=128 * 1024 * 1024,
        ),
    )(tok5, x_emb, y_emb, t_emb.reshape(tau, 1, _D3))

    return out5.reshape(B, tau, N, d)
